# baseline (speedup 1.0000x reference)
import math

import jax
import jax.numpy as jnp
from jax import lax
from jax.experimental import pallas as pl
from jax.experimental.pallas import tpu as pltpu
from jax.experimental.pallas import tpu_sc as plsc

_N = 20000
_L = 16
_TILE = 128
_WCOLS = 10 * _TILE
_NTILES = 157
_CLAMP = _NTILES * _TILE - _WCOLS
assert 15 * _WCOLS >= _CLAMP
_NG = _WCOLS // _L

_SCALE_CLAMP = math.log(224.0 / 8.0)
_BG = -1e8


_HALF = _WCOLS // 2


def _sc_body(d_hbm, a_hbm, o_hbm, d_v, a_v, o_v, sem1, sem2, sem3):
    wid = lax.axis_index("s")
    start = pl.multiple_of(jnp.minimum(wid * _WCOLS, _CLAMP), _TILE)
    i1d = pltpu.async_copy(d_hbm.at[:, pl.ds(start, _HALF)],
                           d_v.at[:, pl.ds(0, _HALF)], sem1)
    i1a = pltpu.async_copy(a_hbm.at[:, pl.ds(start, _HALF)],
                           a_v.at[:, pl.ds(0, _HALF)], sem1)
    i2d = pltpu.async_copy(d_hbm.at[:, pl.ds(start + _HALF, _HALF)],
                           d_v.at[:, pl.ds(_HALF, _HALF)], sem2)
    i2a = pltpu.async_copy(a_hbm.at[:, pl.ds(start + _HALF, _HALF)],
                           a_v.at[:, pl.ds(_HALF, _HALF)], sem2)

    clamp = jnp.full((_L,), _SCALE_CLAMP, jnp.float32)
    bg = jnp.full((_L,), _BG, jnp.float32)

    def _step(g):
        o = g * _L
        dx = d_v[0, pl.ds(o, _L)]
        dy = d_v[1, pl.ds(o, _L)]
        dw = d_v[2, pl.ds(o, _L)]
        dh = d_v[3, pl.ds(o, _L)]
        x1 = a_v[0, pl.ds(o, _L)]
        y1 = a_v[1, pl.ds(o, _L)]
        x2 = a_v[2, pl.ds(o, _L)]
        y2 = a_v[3, pl.ds(o, _L)]
        pw = x2 - x1
        ph = y2 - y1
        px = (x1 + x2) * 0.5
        py = (y1 + y2) * 0.5
        bw2 = jnp.exp(jnp.minimum(dw, clamp)) * pw * 0.5
        bh2 = jnp.exp(jnp.minimum(dh, clamp)) * ph * 0.5
        bx = dx * pw + px
        by = dy * ph + py
        fg = dx != bg
        o_v[0, pl.ds(o, _L)] = jnp.where(fg, bx - bw2, bg)
        o_v[1, pl.ds(o, _L)] = jnp.where(fg, by - bh2, bg)
        o_v[2, pl.ds(o, _L)] = jnp.where(fg, bx + bw2, bg)
        o_v[3, pl.ds(o, _L)] = jnp.where(fg, by + bh2, bg)

    i1d.wait()
    i1a.wait()

    @plsc.parallel_loop(0, _HALF // _L, unroll=2)
    def _loop1(g):
        _step(g)

    o1 = pltpu.async_copy(o_v.at[:, pl.ds(0, _HALF)],
                          o_hbm.at[:, pl.ds(start, _HALF)], sem3)
    i2d.wait()
    i2a.wait()

    @plsc.parallel_loop(_HALF // _L, _NG, unroll=2)
    def _loop2(g):
        _step(g)

    pltpu.sync_copy(o_v.at[:, pl.ds(_HALF, _HALF)],
                    o_hbm.at[:, pl.ds(start + _HALF, _HALF)])
    o1.wait()


_decode = pl.kernel(
    _sc_body,
    out_type=jax.ShapeDtypeStruct((4, _N), jnp.float32),
    mesh=plsc.VectorSubcoreMesh(core_axis_name="c", subcore_axis_name="s",
                                num_cores=1, num_subcores=16),
    compiler_params=pltpu.CompilerParams(
        needs_layout_passes=False,
        skip_device_barrier=True,
        disable_bounds_checks=True,
        disable_semaphore_checks=True,
    ),
    scratch_types=[
        pltpu.VMEM((4, _WCOLS), jnp.float32),
        pltpu.VMEM((4, _WCOLS), jnp.float32),
        pltpu.VMEM((4, _WCOLS), jnp.float32),
        pltpu.SemaphoreType.DMA,
        pltpu.SemaphoreType.DMA,
        pltpu.SemaphoreType.DMA,
    ],
)


def kernel(deltas, anchors):
    return _decode(deltas.T, anchors.T).T
